# Initial kernel scaffold; baseline (speedup 1.0000x reference)
import functools

import jax
import jax.numpy as jnp
from jax import lax
from jax.experimental import pallas as pl
from jax.experimental.pallas import tpu as pltpu
from jax.experimental.pallas import tpu_sc as plsc

N = 10000
E = 320000
D = 128
L = 3
G = 256

NC = 2
NS = 16
NW = NC * NS
CH = 128
NCH = 80
EPW = NCH * CH
EPAD = NW * EPW
APAD = 10240


def _sc_scatter(a_pad, dw, table, src, dst):
  mesh = plsc.VectorSubcoreMesh(core_axis_name="c", subcore_axis_name="s")
  bps = a_pad // (NS * CH)

  @functools.partial(
      pl.kernel,
      out_type=jax.ShapeDtypeStruct((NC, a_pad, dw), jnp.float32),
      mesh=mesh,
      scratch_types=[
          pltpu.VMEM((NCH, CH), jnp.int32),
          pltpu.VMEM((NCH, CH), jnp.int32),
          pltpu.VMEM((2, CH, dw), jnp.float32),
          pltpu.VMEM_SHARED((a_pad, dw), jnp.float32),
          pltpu.SemaphoreType.DMA,
          pltpu.SemaphoreType.DMA,
      ],
  )
  def body(table_r, src_r, dst_r, out_r, src_v, dst_v, rows_v, accum, g0, g1):
    c = lax.axis_index("c")
    s = lax.axis_index("s")
    wid = s * NC + c

    def zrow(i, _):
      def zcol(j, _):
        rows_v[0, i, pl.ds(j * 16, 16)] = jnp.zeros((16,), jnp.float32)
        return 0
      return lax.fori_loop(0, dw // 16, zcol, 0)
    lax.fori_loop(0, CH, zrow, 0)
    for k in range(bps):
      base = (s * bps + k) * CH
      pltpu.sync_copy(rows_v.at[0], accum.at[pl.ds(base, CH)])
    plsc.subcore_barrier()

    pltpu.sync_copy(src_r.at[wid], src_v)
    pltpu.sync_copy(dst_r.at[wid], dst_v)

    pltpu.async_copy(table_r.at[src_v.at[0]], rows_v.at[0], g0)
    pltpu.async_copy(table_r.at[src_v.at[1]], rows_v.at[1], g1)

    def pair(j2, _):
      for b in range(2):
        j = j2 * 2 + b
        sem = g0 if b == 0 else g1
        buf = rows_v.at[b]
        pltpu.make_async_copy(table_r.at[pl.ds(0, CH)], buf, sem).wait()
        pltpu.sync_copy(buf, accum.at[dst_v.at[j]], add=True)
        pltpu.async_copy(table_r.at[src_v.at[j + 2]], buf, sem)
      return 0
    lax.fori_loop(0, NCH // 2 - 1, pair, 0)
    for b in range(2):
      j = NCH - 2 + b
      sem = g0 if b == 0 else g1
      buf = rows_v.at[b]
      pltpu.make_async_copy(table_r.at[pl.ds(0, CH)], buf, sem).wait()
      pltpu.sync_copy(buf, accum.at[dst_v.at[j]], add=True)

    plsc.subcore_barrier()
    for k in range(bps):
      base = (s * bps + k) * CH
      pltpu.sync_copy(accum.at[pl.ds(base, CH)],
                      out_r.at[c].at[pl.ds(base, CH)])

  return body(table, src, dst)


def _embed_body(x_ref, tbl_ref, h_ref):
  x0 = x_ref[:, 0:1]
  x1 = x_ref[:, 1:2] + 120
  col = lax.broadcasted_iota(jnp.int32, (N, 128), 1)
  oh = (col == x0).astype(jnp.float32) + (col == x1).astype(jnp.float32)
  h_ref[...] = jnp.dot(oh, tbl_ref[...], preferred_element_type=jnp.float32)


def _layer_body(last, parts_ref, cnts_ref, h_ref, e1_ref, e2_ref,
                w1_ref, b1_ref, w2_ref, b2_ref, gam_ref, bet_ref, out_ref):
  agg = parts_ref[0, :N, :] + parts_ref[1, :N, :] + h_ref[...]
  cnt = cnts_ref[0, :N, :] + cnts_ref[1, :N, :]
  k6 = lax.broadcasted_iota(jnp.int32, (16, 6), 0)
  a6 = lax.broadcasted_iota(jnp.int32, (16, 6), 1)
  s1 = (a6 == k6 // 3).astype(jnp.float32)
  k3 = lax.broadcasted_iota(jnp.int32, (16, 3), 0)
  a3 = lax.broadcasted_iota(jnp.int32, (16, 3), 1)
  s2 = (a3 == k3 % 3).astype(jnp.float32)
  embc = (jnp.dot(s1, e1_ref[...], preferred_element_type=jnp.float32)
          + jnp.dot(s2, e2_ref[...], preferred_element_type=jnp.float32))
  agg = agg + jnp.dot(cnt, embc, preferred_element_type=jnp.float32)
  agg = agg + e1_ref[4:5, :] + e2_ref[0:1, :]
  hmid = jnp.dot(agg, w1_ref[...], preferred_element_type=jnp.float32)
  hmid = jnp.maximum(hmid + b1_ref[...], 0.0)
  hout = jnp.dot(hmid, w2_ref[...], preferred_element_type=jnp.float32)
  hout = hout + b2_ref[...]
  mean = jnp.mean(hout, axis=0, keepdims=True)
  var = jnp.mean((hout - mean) ** 2, axis=0, keepdims=True)
  y = (hout - mean) * lax.rsqrt(var + 1e-5) * gam_ref[...] + bet_ref[...]
  if not last:
    y = jnp.maximum(y, 0.0)
  out_ref[...] = y


def _pool_body(h_ref, batch_ref, gr_ref):
  row = lax.broadcasted_iota(jnp.int32, (G, N), 0)
  oh = (row == batch_ref[...]).astype(jnp.float32)
  sums = jnp.dot(oh, h_ref[...], preferred_element_type=jnp.float32)
  counts = jnp.sum(oh, axis=1, keepdims=True)
  gr_ref[...] = sums / jnp.maximum(counts, 1.0)


def kernel(x, edge_index, edge_attr, batch, x_emb1, x_emb2, edge_emb1,
           edge_emb2, W1, b1, W2, b2, gamma, beta):
  f32 = jnp.float32

  src = jnp.concatenate([edge_index[0], jnp.zeros((EPAD - E,), jnp.int32)])
  dst = jnp.concatenate([edge_index[1],
                         jnp.full((EPAD - E,), N, jnp.int32)])
  code = edge_attr[:, 0] * 3 + edge_attr[:, 1]
  code = jnp.concatenate([code, jnp.zeros((EPAD - E,), jnp.int32)])
  src = src.reshape(NW, NCH, CH)
  dst = dst.reshape(NW, NCH, CH)
  code = code.reshape(NW, NCH, CH)

  eye16 = jnp.concatenate([jnp.eye(16, dtype=f32),
                           jnp.zeros((112, 16), f32)])
  emb_tbl = jnp.concatenate([x_emb1, x_emb2,
                             jnp.zeros((128 - 123, D), f32)])

  cnts = _sc_scatter(APAD, 16, eye16, code, dst)

  h = pl.pallas_call(
      _embed_body,
      out_shape=jax.ShapeDtypeStruct((N, D), f32),
  )(x, emb_tbl)

  for l in range(L):
    parts = _sc_scatter(APAD, D, h, src, dst)
    h = pl.pallas_call(
        functools.partial(_layer_body, l == L - 1),
        out_shape=jax.ShapeDtypeStruct((N, D), f32),
    )(parts, cnts, h, edge_emb1[l], edge_emb2[l],
      W1[l], b1[l].reshape(1, 2 * D), W2[l], b2[l].reshape(1, D),
      gamma[l].reshape(1, D), beta[l].reshape(1, D))

  graph_rep = pl.pallas_call(
      _pool_body,
      out_shape=jax.ShapeDtypeStruct((G, D), f32),
  )(h, batch.reshape(1, N))

  return (graph_rep, h)


# speedup vs baseline: 5.0802x; 5.0802x over previous
import functools

import jax
import jax.numpy as jnp
from jax import lax
from jax.experimental import pallas as pl
from jax.experimental.pallas import tpu as pltpu
from jax.experimental.pallas import tpu_sc as plsc

N = 10000
E = 320000
D = 128
L = 3
G = 256

NC = 2
NS = 16
NW = NC * NS
EPW = 10240
EPAD = NW * EPW
APAD = 10240


CH = 128
NCH = EPW // CH


def _unpack_row(packed_v, j, half, out_v, b):
  for g in range(CH // 16):
    v = packed_v[j, pl.ds(g * 16, 16)]
    if half == 0:
      v = jnp.bitwise_and(v, jnp.int32(0xFFFF))
    else:
      v = lax.shift_right_logical(v, jnp.int32(16))
    out_v[b, pl.ds(g * 16, 16)] = v


def _sc_scatter(a_pad, table, packed):
  mesh = plsc.VectorSubcoreMesh(core_axis_name="c", subcore_axis_name="s")
  bps = a_pad // (NS * CH)

  @functools.partial(
      pl.kernel,
      out_type=jax.ShapeDtypeStruct((NC, a_pad, D), jnp.float32),
      mesh=mesh,
      scratch_types=[
          pltpu.VMEM((NCH, CH), jnp.int32),
          pltpu.VMEM((2, CH), jnp.int32),
          pltpu.VMEM((2, CH), jnp.int32),
          pltpu.VMEM((2, CH, D), jnp.float32),
          pltpu.VMEM_SHARED((a_pad, D), jnp.float32),
          pltpu.SemaphoreType.DMA,
          pltpu.SemaphoreType.DMA,
      ],
  )
  def body(table_r, packed_r, out_r, packed_v, src_v, dst_v, rows_v, accum,
           g0, g1):
    c = lax.axis_index("c")
    s = lax.axis_index("s")
    wid = s * NC + c

    def zrow(i, _):
      def zcol(j, _):
        rows_v[0, i, pl.ds(j * 16, 16)] = jnp.zeros((16,), jnp.float32)
        return 0
      return lax.fori_loop(0, D // 16, zcol, 0)
    lax.fori_loop(0, CH, zrow, 0)
    for k in range(bps):
      base = (s * bps + k) * CH
      pltpu.sync_copy(rows_v.at[0], accum.at[pl.ds(base, CH)])
    plsc.subcore_barrier()

    pltpu.sync_copy(packed_r.at[wid], packed_v)

    def step(j, _):
      _unpack_row(packed_v, j, 0, src_v, 0)
      pltpu.async_copy(table_r.at[src_v.at[0]], rows_v.at[0], g0).wait()
      _unpack_row(packed_v, j, 1, dst_v, 0)
      pltpu.sync_copy(rows_v.at[0], accum.at[dst_v.at[0]], add=True)
      return 0
    lax.fori_loop(0, NCH, step, 0)

    plsc.subcore_barrier()
    for k in range(bps):
      base = (s * bps + k) * CH
      pltpu.sync_copy(accum.at[pl.ds(base, CH)],
                      out_r.at[c].at[pl.ds(base, CH)])

  return body(table, packed)


def _sc_counts(a_flat, fidx):
  mesh = plsc.VectorSubcoreMesh(core_axis_name="c", subcore_axis_name="s")
  eps = a_flat // NS
  zb = 2048

  @functools.partial(
      pl.kernel,
      out_type=jax.ShapeDtypeStruct((NC, a_flat), jnp.float32),
      mesh=mesh,
      scratch_types=[
          pltpu.VMEM((NCH, CH), jnp.int32),
          pltpu.VMEM((zb,), jnp.float32),
          pltpu.VMEM_SHARED((a_flat,), jnp.float32),
      ],
  )
  def body(fidx_r, out_r, fidx_v, ones_v, accum):
    c = lax.axis_index("c")
    s = lax.axis_index("s")
    wid = s * NC + c

    def zblk(i, _):
      ones_v[pl.ds(i * 16, 16)] = jnp.zeros((16,), jnp.float32)
      return 0
    lax.fori_loop(0, zb // 16, zblk, 0)
    for k in range(eps // zb):
      pltpu.sync_copy(ones_v, accum.at[pl.ds(s * eps + k * zb, zb)])
    plsc.subcore_barrier()

    def oblk(i, _):
      ones_v[pl.ds(i * 16, 16)] = jnp.ones((16,), jnp.float32)
      return 0
    lax.fori_loop(0, CH // 16, oblk, 0)

    pltpu.sync_copy(fidx_r.at[wid], fidx_v)

    def step(j, _):
      pltpu.sync_copy(ones_v.at[pl.ds(0, CH)], accum.at[fidx_v.at[j]],
                      add=True)
      return 0
    lax.fori_loop(0, NCH, step, 0)

    plsc.subcore_barrier()
    pltpu.sync_copy(accum.at[pl.ds(s * eps, eps)],
                    out_r.at[c].at[pl.ds(s * eps, eps)])

  return body(fidx)


def _embed_body(x_ref, tbl_ref, h_ref):
  x0 = x_ref[:, 0:1]
  x1 = x_ref[:, 1:2] + 120
  col = lax.broadcasted_iota(jnp.int32, (N, 128), 1)
  oh = (col == x0).astype(jnp.float32) + (col == x1).astype(jnp.float32)
  h_ref[...] = jnp.dot(oh, tbl_ref[...], preferred_element_type=jnp.float32,
                precision=lax.Precision.HIGHEST)


def _layer_body(last, parts_ref, cnts_ref, h_ref, e1_ref, e2_ref,
                w1_ref, b1_ref, w2_ref, b2_ref, gam_ref, bet_ref, out_ref):
  agg = parts_ref[0, :N, :] + parts_ref[1, :N, :] + h_ref[...]
  cnt = cnts_ref[0, :N, :] + cnts_ref[1, :N, :]
  k6 = lax.broadcasted_iota(jnp.int32, (16, 6), 0)
  a6 = lax.broadcasted_iota(jnp.int32, (16, 6), 1)
  s1 = (a6 == k6 // 3).astype(jnp.float32)
  k3 = lax.broadcasted_iota(jnp.int32, (16, 3), 0)
  a3 = lax.broadcasted_iota(jnp.int32, (16, 3), 1)
  s2 = (a3 == k3 % 3).astype(jnp.float32)
  embc = (jnp.dot(s1, e1_ref[...], preferred_element_type=jnp.float32,
                precision=lax.Precision.HIGHEST)
          + jnp.dot(s2, e2_ref[...], preferred_element_type=jnp.float32,
                precision=lax.Precision.HIGHEST))
  agg = agg + jnp.dot(cnt, embc, preferred_element_type=jnp.float32,
                precision=lax.Precision.HIGHEST)
  agg = agg + e1_ref[4:5, :] + e2_ref[0:1, :]
  hmid = jnp.dot(agg, w1_ref[...], preferred_element_type=jnp.float32)
  hmid = jnp.maximum(hmid + b1_ref[...], 0.0)
  hout = jnp.dot(hmid, w2_ref[...], preferred_element_type=jnp.float32)
  hout = hout + b2_ref[...]
  mean = jnp.mean(hout, axis=0, keepdims=True)
  var = jnp.mean((hout - mean) ** 2, axis=0, keepdims=True)
  y = (hout - mean) * lax.rsqrt(var + 1e-5) * gam_ref[...] + bet_ref[...]
  if not last:
    y = jnp.maximum(y, 0.0)
  out_ref[...] = y


def _pool_body(h_ref, batch_ref, gr_ref):
  row = lax.broadcasted_iota(jnp.int32, (G, N), 0)
  oh = (row == batch_ref[...]).astype(jnp.float32)
  sums = jnp.dot(oh, h_ref[...], preferred_element_type=jnp.float32,
                precision=lax.Precision.HIGHEST)
  counts = jnp.sum(oh, axis=1, keepdims=True)
  gr_ref[...] = sums / jnp.maximum(counts, 1.0)


def kernel(x, edge_index, edge_attr, batch, x_emb1, x_emb2, edge_emb1,
           edge_emb2, W1, b1, W2, b2, gamma, beta):
  f32 = jnp.float32

  src = jnp.concatenate([edge_index[0], jnp.zeros((EPAD - E,), jnp.int32)])
  dst = jnp.concatenate([edge_index[1],
                         jnp.full((EPAD - E,), N, jnp.int32)])
  code = edge_attr[:, 0] * 3 + edge_attr[:, 1]
  code = jnp.concatenate([code, jnp.zeros((EPAD - E,), jnp.int32)])
  packed = (src | (dst << 16)).reshape(NW, NCH, CH)
  fidx = (dst * 16 + code).reshape(NW, NCH, CH)

  emb_tbl = jnp.concatenate([x_emb1, x_emb2,
                             jnp.zeros((128 - 123, D), f32)])

  cnts = _sc_counts(APAD * 16, fidx).reshape(NC, APAD, 16)

  h = pl.pallas_call(
      _embed_body,
      out_shape=jax.ShapeDtypeStruct((N, D), f32),
  )(x, emb_tbl)

  for l in range(L):
    parts = _sc_scatter(APAD, h, packed)
    h = pl.pallas_call(
        functools.partial(_layer_body, l == L - 1),
        out_shape=jax.ShapeDtypeStruct((N, D), f32),
    )(parts, cnts, h, edge_emb1[l], edge_emb2[l],
      W1[l], b1[l].reshape(1, 2 * D), W2[l], b2[l].reshape(1, D),
      gamma[l].reshape(1, D), beta[l].reshape(1, D))

  graph_rep = pl.pallas_call(
      _pool_body,
      out_shape=jax.ShapeDtypeStruct((G, D), f32),
  )(h, batch.reshape(1, N))

  return (graph_rep, h)


# speedup vs baseline: 5.5181x; 1.0862x over previous
import functools

import jax
import jax.numpy as jnp
from jax import lax
from jax.experimental import pallas as pl
from jax.experimental.pallas import tpu as pltpu
from jax.experimental.pallas import tpu_sc as plsc

N = 10000
E = 320000
D = 128
L = 3
G = 256

NC = 2
NS = 16
NW = NC * NS
EPW = 10240
EPAD = NW * EPW
APAD = 10240


CH = 128
NCH = EPW // CH


def _unpack_row(packed_v, j, half, out_v, b):
  for g in range(CH // 16):
    v = packed_v[j, pl.ds(g * 16, 16)]
    if half == 0:
      v = jnp.bitwise_and(v, jnp.int32(0xFFFF))
    else:
      v = lax.shift_right_logical(v, jnp.int32(16))
    out_v[b, pl.ds(g * 16, 16)] = v


def _sc_scatter(a_pad, table, packed):
  mesh = plsc.VectorSubcoreMesh(core_axis_name="c", subcore_axis_name="s")
  bps = a_pad // (NS * CH)

  @functools.partial(
      pl.kernel,
      out_type=jax.ShapeDtypeStruct((NC, a_pad, D), jnp.float32),
      mesh=mesh,
      scratch_types=[
          pltpu.VMEM((NCH, CH), jnp.int32),
          pltpu.VMEM((2, CH), jnp.int32),
          pltpu.VMEM((2, CH), jnp.int32),
          pltpu.VMEM((2, CH, D), jnp.float32),
          pltpu.VMEM_SHARED((a_pad, D), jnp.float32),
          pltpu.SemaphoreType.DMA,
          pltpu.SemaphoreType.DMA,
      ],
  )
  def body(table_r, packed_r, out_r, packed_v, src_v, dst_v, rows_v, accum,
           g0, g1):
    c = lax.axis_index("c")
    s = lax.axis_index("s")
    wid = s * NC + c

    def zrow(i, _):
      def zcol(j, _):
        rows_v[0, i, pl.ds(j * 16, 16)] = jnp.zeros((16,), jnp.float32)
        return 0
      return lax.fori_loop(0, D // 16, zcol, 0)
    lax.fori_loop(0, CH, zrow, 0)
    for k in range(bps):
      base = (s * bps + k) * CH
      pltpu.sync_copy(rows_v.at[0], accum.at[pl.ds(base, CH)])
    plsc.subcore_barrier()

    pltpu.sync_copy(packed_r.at[wid], packed_v)

    for b in range(2):
      _unpack_row(packed_v, b, 0, src_v, b)
      sem = g0 if b == 0 else g1
      pltpu.async_copy(table_r.at[src_v.at[b]], rows_v.at[b], sem)

    def pair(j2, _):
      for b in range(2):
        j = j2 * 2 + b
        sem = g0 if b == 0 else g1
        buf = rows_v.at[b]
        pltpu.make_async_copy(table_r.at[pl.ds(0, CH)], buf, sem).wait()
        _unpack_row(packed_v, j, 1, dst_v, b)
        pltpu.sync_copy(buf, accum.at[dst_v.at[b]], add=True)
        _unpack_row(packed_v, j + 2, 0, src_v, b)
        pltpu.async_copy(table_r.at[src_v.at[b]], buf, sem)
      return 0
    lax.fori_loop(0, NCH // 2 - 1, pair, 0)
    for b in range(2):
      j = NCH - 2 + b
      sem = g0 if b == 0 else g1
      buf = rows_v.at[b]
      pltpu.make_async_copy(table_r.at[pl.ds(0, CH)], buf, sem).wait()
      _unpack_row(packed_v, j, 1, dst_v, b)
      pltpu.sync_copy(buf, accum.at[dst_v.at[b]], add=True)

    plsc.subcore_barrier()
    for k in range(bps):
      base = (s * bps + k) * CH
      pltpu.sync_copy(accum.at[pl.ds(base, CH)],
                      out_r.at[c].at[pl.ds(base, CH)])

  return body(table, packed)


def _sc_counts(a_flat, fidx):
  mesh = plsc.VectorSubcoreMesh(core_axis_name="c", subcore_axis_name="s")
  eps = a_flat // NS
  zb = 2048

  @functools.partial(
      pl.kernel,
      out_type=jax.ShapeDtypeStruct((NC, a_flat), jnp.float32),
      mesh=mesh,
      scratch_types=[
          pltpu.VMEM((NCH, CH), jnp.int32),
          pltpu.VMEM((zb,), jnp.float32),
          pltpu.VMEM_SHARED((a_flat,), jnp.float32),
      ],
  )
  def body(fidx_r, out_r, fidx_v, ones_v, accum):
    c = lax.axis_index("c")
    s = lax.axis_index("s")
    wid = s * NC + c

    def zblk(i, _):
      ones_v[pl.ds(i * 16, 16)] = jnp.zeros((16,), jnp.float32)
      return 0
    lax.fori_loop(0, zb // 16, zblk, 0)
    for k in range(eps // zb):
      pltpu.sync_copy(ones_v, accum.at[pl.ds(s * eps + k * zb, zb)])
    plsc.subcore_barrier()

    def oblk(i, _):
      ones_v[pl.ds(i * 16, 16)] = jnp.ones((16,), jnp.float32)
      return 0
    lax.fori_loop(0, CH // 16, oblk, 0)

    pltpu.sync_copy(fidx_r.at[wid], fidx_v)

    def step(j, _):
      pltpu.sync_copy(ones_v.at[pl.ds(0, CH)], accum.at[fidx_v.at[j]],
                      add=True)
      return 0
    lax.fori_loop(0, NCH, step, 0)

    plsc.subcore_barrier()
    pltpu.sync_copy(accum.at[pl.ds(s * eps, eps)],
                    out_r.at[c].at[pl.ds(s * eps, eps)])

  return body(fidx)


def _embed_body(x_ref, tbl_ref, h_ref):
  x0 = x_ref[:, 0:1]
  x1 = x_ref[:, 1:2] + 120
  col = lax.broadcasted_iota(jnp.int32, (N, 128), 1)
  oh = (col == x0).astype(jnp.float32) + (col == x1).astype(jnp.float32)
  h_ref[...] = jnp.dot(oh, tbl_ref[...], preferred_element_type=jnp.float32,
                precision=lax.Precision.HIGHEST)


def _layer_body(last, parts_ref, cnts_ref, h_ref, e1_ref, e2_ref,
                w1_ref, b1_ref, w2_ref, b2_ref, gam_ref, bet_ref, out_ref):
  agg = parts_ref[0, :N, :] + parts_ref[1, :N, :] + h_ref[...]
  cnt = cnts_ref[0, :N, :] + cnts_ref[1, :N, :]
  k6 = lax.broadcasted_iota(jnp.int32, (16, 6), 0)
  a6 = lax.broadcasted_iota(jnp.int32, (16, 6), 1)
  s1 = (a6 == k6 // 3).astype(jnp.float32)
  k3 = lax.broadcasted_iota(jnp.int32, (16, 3), 0)
  a3 = lax.broadcasted_iota(jnp.int32, (16, 3), 1)
  s2 = (a3 == k3 % 3).astype(jnp.float32)
  embc = (jnp.dot(s1, e1_ref[...], preferred_element_type=jnp.float32,
                precision=lax.Precision.HIGHEST)
          + jnp.dot(s2, e2_ref[...], preferred_element_type=jnp.float32,
                precision=lax.Precision.HIGHEST))
  agg = agg + jnp.dot(cnt, embc, preferred_element_type=jnp.float32,
                precision=lax.Precision.HIGHEST)
  agg = agg + e1_ref[4:5, :] + e2_ref[0:1, :]
  hmid = jnp.dot(agg, w1_ref[...], preferred_element_type=jnp.float32)
  hmid = jnp.maximum(hmid + b1_ref[...], 0.0)
  hout = jnp.dot(hmid, w2_ref[...], preferred_element_type=jnp.float32)
  hout = hout + b2_ref[...]
  mean = jnp.mean(hout, axis=0, keepdims=True)
  var = jnp.mean((hout - mean) ** 2, axis=0, keepdims=True)
  y = (hout - mean) * lax.rsqrt(var + 1e-5) * gam_ref[...] + bet_ref[...]
  if not last:
    y = jnp.maximum(y, 0.0)
  out_ref[...] = y


def _pool_body(h_ref, batch_ref, gr_ref):
  row = lax.broadcasted_iota(jnp.int32, (G, N), 0)
  oh = (row == batch_ref[...]).astype(jnp.float32)
  sums = jnp.dot(oh, h_ref[...], preferred_element_type=jnp.float32,
                precision=lax.Precision.HIGHEST)
  counts = jnp.sum(oh, axis=1, keepdims=True)
  gr_ref[...] = sums / jnp.maximum(counts, 1.0)


def kernel(x, edge_index, edge_attr, batch, x_emb1, x_emb2, edge_emb1,
           edge_emb2, W1, b1, W2, b2, gamma, beta):
  f32 = jnp.float32

  src = jnp.concatenate([edge_index[0], jnp.zeros((EPAD - E,), jnp.int32)])
  dst = jnp.concatenate([edge_index[1],
                         jnp.full((EPAD - E,), N, jnp.int32)])
  code = edge_attr[:, 0] * 3 + edge_attr[:, 1]
  code = jnp.concatenate([code, jnp.zeros((EPAD - E,), jnp.int32)])
  packed = (src | (dst << 16)).reshape(NW, NCH, CH)
  fidx = (dst * 16 + code).reshape(NW, NCH, CH)

  emb_tbl = jnp.concatenate([x_emb1, x_emb2,
                             jnp.zeros((128 - 123, D), f32)])

  cnts = _sc_counts(APAD * 16, fidx).reshape(NC, APAD, 16)

  h = pl.pallas_call(
      _embed_body,
      out_shape=jax.ShapeDtypeStruct((N, D), f32),
  )(x, emb_tbl)

  for l in range(L):
    parts = _sc_scatter(APAD, h, packed)
    h = pl.pallas_call(
        functools.partial(_layer_body, l == L - 1),
        out_shape=jax.ShapeDtypeStruct((N, D), f32),
    )(parts, cnts, h, edge_emb1[l], edge_emb2[l],
      W1[l], b1[l].reshape(1, 2 * D), W2[l], b2[l].reshape(1, D),
      gamma[l].reshape(1, D), beta[l].reshape(1, D))

  graph_rep = pl.pallas_call(
      _pool_body,
      out_shape=jax.ShapeDtypeStruct((G, D), f32),
  )(h, batch.reshape(1, N))

  return (graph_rep, h)
